# baseline (speedup 1.0000x reference)
import functools

import jax
import jax.numpy as jnp
from jax import lax
from jax.experimental import pallas as pl
from jax.experimental.pallas import tpu as pltpu
from jax.experimental.pallas import tpu_sc as plsc

MAX_LEN = 4096
D_MODEL = 1024
BATCH = 4

NUM_CORES = 2
NUM_SUBCORES = 16
NUM_WORKERS = NUM_CORES * NUM_SUBCORES
ROWS_PER_WORKER = MAX_LEN // NUM_WORKERS
CHUNK_ROWS = 32
NUM_CHUNKS = ROWS_PER_WORKER // CHUNK_ROWS


@functools.partial(jax.jit, static_argnames=())
def _broadcast_table(emb_weight):
    mesh = plsc.VectorSubcoreMesh(core_axis_name="c", subcore_axis_name="s")

    @functools.partial(
        pl.kernel,
        mesh=mesh,
        out_type=jax.ShapeDtypeStruct((BATCH, MAX_LEN, D_MODEL), jnp.float32),
        scratch_types=[
            pltpu.VMEM((CHUNK_ROWS, D_MODEL), jnp.float32),
            pltpu.VMEM((CHUNK_ROWS, D_MODEL), jnp.float32),
            pltpu.SemaphoreType.DMA,
            pltpu.SemaphoreType.DMA,
            pltpu.SemaphoreType.DMA,
            pltpu.SemaphoreType.DMA,
        ],
    )
    def k(table_hbm, out_hbm, buf0, buf1, rsem0, rsem1, wsem0, wsem1):
        wid = lax.axis_index("s") * NUM_CORES + lax.axis_index("c")
        base = wid * ROWS_PER_WORKER
        bufs = (buf0, buf1)
        rsems = (rsem0, rsem1)
        wsems = (wsem0, wsem1)

        reads = [None] * NUM_CHUNKS
        writes = [[] for _ in range(NUM_CHUNKS)]
        reads[0] = pltpu.make_async_copy(
            table_hbm.at[pl.ds(base, CHUNK_ROWS), :], bufs[0], rsems[0])
        reads[0].start()
        for c in range(NUM_CHUNKS):
            cur = c % 2
            nxt = (c + 1) % 2
            if c + 1 < NUM_CHUNKS:
                for h in (writes[c - 1] if c >= 1 else []):
                    h.wait()
                r0 = base + (c + 1) * CHUNK_ROWS
                reads[c + 1] = pltpu.make_async_copy(
                    table_hbm.at[pl.ds(r0, CHUNK_ROWS), :], bufs[nxt], rsems[nxt])
                reads[c + 1].start()
            reads[c].wait()
            for b in range(BATCH):
                h = pltpu.make_async_copy(
                    bufs[cur],
                    out_hbm.at[b, pl.ds(base + c * CHUNK_ROWS, CHUNK_ROWS), :],
                    wsems[cur])
                h.start()
                writes[c].append(h)
        for h in writes[NUM_CHUNKS - 2] + writes[NUM_CHUNKS - 1]:
            h.wait()

    return k(emb_weight)


def kernel(x, emb_weight):
    del x
    return _broadcast_table(emb_weight)


# speedup vs baseline: 1.0125x; 1.0125x over previous
import functools

import jax
import jax.numpy as jnp
from jax import lax
from jax.experimental import pallas as pl
from jax.experimental.pallas import tpu as pltpu
from jax.experimental.pallas import tpu_sc as plsc

MAX_LEN = 4096
D_MODEL = 1024
BATCH = 4

NUM_CORES = 2
NUM_SUBCORES = 16
NUM_WORKERS = NUM_CORES * NUM_SUBCORES
ROWS_PER_WORKER = MAX_LEN // NUM_WORKERS
CHUNK_ROWS = 64
NUM_CHUNKS = ROWS_PER_WORKER // CHUNK_ROWS


@functools.partial(jax.jit, static_argnames=())
def _broadcast_table(emb_weight):
    mesh = plsc.VectorSubcoreMesh(core_axis_name="c", subcore_axis_name="s")

    @functools.partial(
        pl.kernel,
        mesh=mesh,
        out_type=jax.ShapeDtypeStruct((BATCH, MAX_LEN, D_MODEL), jnp.float32),
        scratch_types=[pltpu.VMEM((CHUNK_ROWS, D_MODEL), jnp.float32)],
    )
    def k(table_hbm, out_hbm, buf):
        wid = lax.axis_index("s") * NUM_CORES + lax.axis_index("c")
        base = wid * ROWS_PER_WORKER

        def chunk_body(c, carry):
            r0 = base + c * CHUNK_ROWS
            pltpu.sync_copy(table_hbm.at[pl.ds(r0, CHUNK_ROWS), :], buf)

            def batch_body(b, carry2):
                pltpu.sync_copy(buf, out_hbm.at[b, pl.ds(r0, CHUNK_ROWS), :])
                return carry2

            return lax.fori_loop(0, BATCH, batch_body, carry)

        lax.fori_loop(0, NUM_CHUNKS, chunk_body, 0)

    return k(emb_weight)


def kernel(x, emb_weight):
    del x
    return _broadcast_table(emb_weight)


# speedup vs baseline: 1.1790x; 1.1644x over previous
import functools

import jax
import jax.numpy as jnp
from jax.experimental import pallas as pl
from jax.experimental.pallas import tpu as pltpu

MAX_LEN = 4096
D_MODEL = 1024
BATCH = 4

BLOCK_ROWS = 512
NUM_BLOCKS = MAX_LEN // BLOCK_ROWS


def _copy_body(table_ref, out_ref):
    out_ref[0] = table_ref[...]


@jax.jit
def _broadcast_table(emb_weight):
    return pl.pallas_call(
        _copy_body,
        grid=(NUM_BLOCKS, BATCH),
        in_specs=[
            pl.BlockSpec((BLOCK_ROWS, D_MODEL), lambda i, b: (i, 0)),
        ],
        out_specs=pl.BlockSpec((1, BLOCK_ROWS, D_MODEL), lambda i, b: (b, i, 0)),
        out_shape=jax.ShapeDtypeStruct((BATCH, MAX_LEN, D_MODEL), jnp.float32),
    )(emb_weight)


def kernel(x, emb_weight):
    del x
    return _broadcast_table(emb_weight)


# speedup vs baseline: 1.5401x; 1.3062x over previous
import functools

import jax
import jax.numpy as jnp
from jax.experimental import pallas as pl
from jax.experimental.pallas import tpu as pltpu

MAX_LEN = 4096
D_MODEL = 1024
BATCH = 4

CHUNK_ROWS = 512
NUM_CHUNKS = MAX_LEN // CHUNK_ROWS


def _dma_body(table_hbm, out_hbm, buf0, buf1, rsems, wsems):
    bufs = (buf0, buf1)

    def read(c):
        h = pltpu.make_async_copy(
            table_hbm.at[pl.ds(c * CHUNK_ROWS, CHUNK_ROWS), :],
            bufs[c % 2], rsems.at[c % 2])
        h.start()
        return h

    def write(c, b):
        h = pltpu.make_async_copy(
            bufs[c % 2],
            out_hbm.at[b, pl.ds(c * CHUNK_ROWS, CHUNK_ROWS), :],
            wsems.at[c % 2])
        h.start()
        return h

    reads = [None] * NUM_CHUNKS
    writes = [[] for _ in range(NUM_CHUNKS)]
    reads[0] = read(0)
    for c in range(NUM_CHUNKS):
        if c + 1 < NUM_CHUNKS:
            for h in (writes[c - 1] if c >= 1 else []):
                h.wait()
            reads[c + 1] = read(c + 1)
        reads[c].wait()
        writes[c] = [write(c, b) for b in range(BATCH)]
    for h in writes[NUM_CHUNKS - 2] + writes[NUM_CHUNKS - 1]:
        h.wait()


@jax.jit
def _broadcast_table(emb_weight):
    return pl.pallas_call(
        _dma_body,
        in_specs=[pl.BlockSpec(memory_space=pltpu.MemorySpace.HBM)],
        out_specs=pl.BlockSpec(memory_space=pltpu.MemorySpace.HBM),
        out_shape=jax.ShapeDtypeStruct((BATCH, MAX_LEN, D_MODEL), jnp.float32),
        scratch_shapes=[
            pltpu.VMEM((CHUNK_ROWS, D_MODEL), jnp.float32),
            pltpu.VMEM((CHUNK_ROWS, D_MODEL), jnp.float32),
            pltpu.SemaphoreType.DMA((2,)),
            pltpu.SemaphoreType.DMA((2,)),
        ],
    )(emb_weight)


def kernel(x, emb_weight):
    del x
    return _broadcast_table(emb_weight)


# speedup vs baseline: 1.8357x; 1.1919x over previous
import functools

import jax
import jax.numpy as jnp
from jax.experimental import pallas as pl
from jax.experimental.pallas import tpu as pltpu

MAX_LEN = 4096
D_MODEL = 1024
BATCH = 4

CHUNK_ROWS = 512
NUM_CHUNKS = MAX_LEN // CHUNK_ROWS


def _dma_body(table_hbm, out_hbm, buf, rsems, wsems):
    def read(c):
        h = pltpu.make_async_copy(
            table_hbm.at[pl.ds(c * CHUNK_ROWS, CHUNK_ROWS), :],
            buf.at[c], rsems.at[c])
        h.start()
        return h

    def write(c, b):
        h = pltpu.make_async_copy(
            buf.at[c],
            out_hbm.at[b, pl.ds(c * CHUNK_ROWS, CHUNK_ROWS), :],
            wsems.at[c])
        h.start()
        return h

    reads = [read(c) for c in range(NUM_CHUNKS)]
    writes = []
    for c in range(NUM_CHUNKS):
        reads[c].wait()
        writes += [write(c, b) for b in range(BATCH)]
    for h in writes:
        h.wait()


@jax.jit
def _broadcast_table(emb_weight):
    return pl.pallas_call(
        _dma_body,
        in_specs=[pl.BlockSpec(memory_space=pltpu.MemorySpace.HBM)],
        out_specs=pl.BlockSpec(memory_space=pltpu.MemorySpace.HBM),
        out_shape=jax.ShapeDtypeStruct((BATCH, MAX_LEN, D_MODEL), jnp.float32),
        scratch_shapes=[
            pltpu.VMEM((NUM_CHUNKS, CHUNK_ROWS, D_MODEL), jnp.float32),
            pltpu.SemaphoreType.DMA((NUM_CHUNKS,)),
            pltpu.SemaphoreType.DMA((NUM_CHUNKS,)),
        ],
    )(emb_weight)


def kernel(x, emb_weight):
    del x
    return _broadcast_table(emb_weight)
